# baseline (speedup 1.0000x reference)
import jax
import jax.numpy as jnp
from jax.experimental import pallas as pl
from jax.experimental.pallas import tpu as pltpu


def _make_pos_kernel(b, d, h, w):

    def _pos_kernel(row_ref, col_ref, out_ref, scratch_ref, sem):
        col_t = col_ref[0:w, :].T
        row_t = row_ref[0:h, :].T
        scratch_ref[0:d] = jnp.broadcast_to(col_t[:, None, :], (d, h, w))
        scratch_ref[d:2 * d] = jnp.broadcast_to(row_t[:, :, None], (d, h, w))
        copies = [
            pltpu.make_async_copy(scratch_ref, out_ref.at[i], sem.at[i])
            for i in range(b)
        ]
        for c in copies:
            c.start()
        for c in copies:
            c.wait()

    return _pos_kernel


def kernel(tensor, row_embed, col_embed):
    b = tensor.shape[0]
    h, w = tensor.shape[-2], tensor.shape[-1]
    d = row_embed.shape[1]

    return pl.pallas_call(
        _make_pos_kernel(b, d, h, w),
        in_specs=[
            pl.BlockSpec(row_embed.shape, lambda: (0, 0)),
            pl.BlockSpec(col_embed.shape, lambda: (0, 0)),
        ],
        out_specs=pl.BlockSpec(memory_space=pl.ANY),
        out_shape=jax.ShapeDtypeStruct((b, 2 * d, h, w), jnp.float32),
        scratch_shapes=[
            pltpu.VMEM((2 * d, h, w), jnp.float32),
            pltpu.SemaphoreType.DMA((b,)),
        ],
    )(row_embed, col_embed)
